# baseline (speedup 1.0000x reference)
import functools

import jax
import jax.numpy as jnp
from jax import lax
from jax.experimental import pallas as pl
from jax.experimental.pallas import tpu as pltpu
from jax.experimental.pallas import tpu_sc as plsc

VOCAB = 1000000
MAXLEN = 200
EMBED = 32
BATCH = 4096

NC = 2
NS = 16
NW = NC * NS

T = BATCH * MAXLEN
TPW = T // NW
R = 4
C = R * MAXLEN
NCHUNK = TPW // C
SUB = 100
SPC = C // SUB
IDX_ROWS_PER_W = TPW // SUB

NBT = BATCH // 128
NTILE = MAXLEN * NBT
TILES_PW = NTILE // NW


def _make_gather():
    mesh = plsc.VectorSubcoreMesh(core_axis_name="c", subcore_axis_name="s")

    @functools.partial(
        pl.kernel,
        out_type=jax.ShapeDtypeStruct((T, EMBED), jnp.float32),
        mesh=mesh,
        scratch_types=[
            pltpu.VMEM((IDX_ROWS_PER_W, SUB), jnp.int32),
            pltpu.VMEM((C, EMBED), jnp.float32),
            pltpu.SemaphoreType.DMA,
        ],
        compiler_params=pltpu.CompilerParams(use_tc_tiling_on_sc=False),
    )
    def emb(x_hbm, tok_hbm, out_hbm, idx_v, rows_v, sem):
        cid = lax.axis_index("c")
        sid = lax.axis_index("s")
        wid = sid * NC + cid

        pltpu.sync_copy(
            x_hbm.at[pl.ds(wid * IDX_ROWS_PER_W, IDX_ROWS_PER_W)], idx_v)

        def do_chunk(g, carry):
            copies = []
            for si in range(SPC):
                row = g * SPC + si
                copies.append(pltpu.async_copy(
                    tok_hbm.at[idx_v.at[row]],
                    rows_v.at[pl.ds(si * SUB, SUB)],
                    sem))
            for cp in copies:
                cp.wait()
            pltpu.sync_copy(
                rows_v, out_hbm.at[pl.ds(wid * TPW + g * C, C)])
            return carry

        lax.fori_loop(0, NCHUNK, do_chunk, 0)

    return emb


def _make_posadd_transpose():
    mesh = plsc.VectorSubcoreMesh(core_axis_name="c", subcore_axis_name="s")

    @functools.partial(
        pl.kernel,
        out_type=jax.ShapeDtypeStruct((MAXLEN, 4, 32 * 8 * 128), jnp.float32),
        mesh=mesh,
        scratch_types=[
            pltpu.VMEM((2, 128, EMBED), jnp.float32),
            pltpu.VMEM((2, 4, 1024), jnp.float32),
            pltpu.VMEM((MAXLEN, EMBED), jnp.float32),
            pltpu.VMEM((2, 128), jnp.int32),
            pltpu.SemaphoreType.DMA,
            pltpu.SemaphoreType.DMA,
            pltpu.SemaphoreType.DMA,
            pltpu.SemaphoreType.DMA,
        ],
        compiler_params=pltpu.CompilerParams(
            use_tc_tiling_on_sc=False, needs_layout_passes=False),
    )
    def ctr(oc_hbm, pos_hbm, o5_hbm, tile_v, blk_v, pos_v, idx_v,
            gs0, gs1, os0, os1):
        cid = lax.axis_index("c")
        sid = lax.axis_index("s")
        wid = sid * NC + cid
        base = wid * TILES_PW

        gsems = [gs0, gs1]
        osems = [os0, os1]
        iota16 = lax.iota(jnp.int32, 16)
        vec200 = iota16 * MAXLEN

        pltpu.sync_copy(pos_hbm, pos_v)

        def lbt(i):
            tau = base + i
            return tau // NBT, tau % NBT

        def fire(i, s):
            l, bt = lbt(i)
            t0 = bt * (128 * MAXLEN) + l
            for jg in range(8):
                idx_v[s, pl.ds(16 * jg, 16)] = vec200 + (
                    t0 + 16 * MAXLEN * jg)
            pltpu.async_copy(
                oc_hbm.at[idx_v.at[s]],
                tile_v.at[s], gsems[s])

        def drain_read(s):
            pltpu.make_async_copy(
                oc_hbm.at[idx_v.at[s]],
                tile_v.at[s], gsems[s]).wait()

        def transpose_add(i, s):
            l, _ = lbt(i)
            p0 = pos_v[l, pl.ds(0, 16)]
            p1 = pos_v[l, pl.ds(16, 16)]
            for e in range(EMBED):
                p = (p0 if e < 16 else p1)[e % 16]
                evec = jnp.full((16,), e, jnp.int32)
                for j in range(8):
                    val = plsc.load_gather(
                        tile_v.at[s], [iota16 + 16 * j, evec])
                    blk_v[s, e // 8, pl.ds((e % 8) * 128 + 16 * j, 16)] = (
                        val + p)

        def store(i, s):
            l, bt = lbt(i)
            pltpu.async_copy(
                blk_v.at[s],
                o5_hbm.at[l, pl.ds(0, 4), pl.ds(bt * 1024, 1024)], osems[s])

        def drain_store(s):
            pltpu.make_async_copy(
                blk_v.at[s],
                o5_hbm.at[0, pl.ds(0, 4), pl.ds(0, 1024)], osems[s]).wait()

        fire(0, 0)

        def tile_loop(h, carry):
            for s in range(2):
                g = 2 * h + s
                ns = 1 - s

                @pl.when(g + 1 < TILES_PW)
                def _(g=g, ns=ns):
                    @pl.when(g >= 1)
                    def _():
                        drain_store(ns)
                    fire(g + 1, ns)

                drain_read(s)
                transpose_add(g, s)
                store(g, s)
            return carry

        lax.fori_loop(0, TILES_PW // 2, tile_loop, 0)
        drain_store(0)
        drain_store(1)

    return ctr


_emb = _make_gather()
_ctr = _make_posadd_transpose()


def kernel(x, token_table, pos_table):
    b, l = x.shape
    x2 = x.reshape(T // SUB, SUB).astype(jnp.int32)
    tok_rows = _emb(x2, token_table)
    o5 = _ctr(tok_rows, pos_table)
    out = (o5.reshape(MAXLEN, 4, 32, 8, 128)
           .transpose(2, 4, 0, 1, 3)
           .reshape(BATCH, MAXLEN, EMBED))
    return out


# speedup vs baseline: 1.3213x; 1.3213x over previous
import functools

import jax
import jax.numpy as jnp
from jax import lax
from jax.experimental import pallas as pl
from jax.experimental.pallas import tpu as pltpu
from jax.experimental.pallas import tpu_sc as plsc

VOCAB = 1000000
MAXLEN = 200
EMBED = 32
BATCH = 4096

NC = 2
NS = 16
NW = NC * NS

T = BATCH * MAXLEN
TPW = T // NW
R = 4
C = R * MAXLEN
NCHUNK = TPW // C
SUB = 100
SPC = C // SUB
IDX_ROWS_PER_W = TPW // SUB


def _make_kernel():
    mesh = plsc.VectorSubcoreMesh(core_axis_name="c", subcore_axis_name="s")

    @functools.partial(
        pl.kernel,
        out_type=jax.ShapeDtypeStruct((T, EMBED), jnp.float32),
        mesh=mesh,
        scratch_types=[
            pltpu.VMEM((IDX_ROWS_PER_W, SUB), jnp.int32),
            pltpu.VMEM((C, EMBED), jnp.float32),
            pltpu.VMEM((MAXLEN, EMBED), jnp.float32),
            pltpu.SemaphoreType.DMA,
        ],
        compiler_params=pltpu.CompilerParams(use_tc_tiling_on_sc=False),
    )
    def emb(x_hbm, tok_hbm, pos_hbm, out_hbm, idx_v, rows_v, pos_v, sem):
        cid = lax.axis_index("c")
        sid = lax.axis_index("s")
        wid = sid * NC + cid

        pltpu.sync_copy(pos_hbm, pos_v)
        pltpu.sync_copy(
            x_hbm.at[pl.ds(wid * IDX_ROWS_PER_W, IDX_ROWS_PER_W)], idx_v)

        def do_chunk(g, carry):
            copies = []
            for si in range(SPC):
                row = g * SPC + si
                copies.append(pltpu.async_copy(
                    tok_hbm.at[idx_v.at[row]],
                    rows_v.at[pl.ds(si * SUB, SUB)],
                    sem))
            for cp in copies:
                cp.wait()

            def add_l(l, c2):
                p0 = pos_v[l, pl.ds(0, 16)]
                p1 = pos_v[l, pl.ds(16, 16)]
                for r in range(R):
                    t = r * MAXLEN + l
                    rows_v[t, pl.ds(0, 16)] += p0
                    rows_v[t, pl.ds(16, 16)] += p1
                return c2

            lax.fori_loop(0, MAXLEN, add_l, 0, unroll=2)

            pltpu.sync_copy(
                rows_v, out_hbm.at[pl.ds(wid * TPW + g * C, C)])
            return carry

        lax.fori_loop(0, NCHUNK, do_chunk, 0)

    return emb


_emb = _make_kernel()


def kernel(x, token_table, pos_table):
    b, l = x.shape
    x2 = x.reshape(T // SUB, SUB).astype(jnp.int32)
    out = _emb(x2, token_table, pos_table)
    return out.reshape(b, l, EMBED)
